# speedup vs baseline: 3.0679x; 3.0679x over previous
import functools

import jax
import jax.numpy as jnp
from jax.experimental import pallas as pl
from jax.experimental.pallas import tpu as pltpu

_E_IND, _E_SH, _TOPK = 8, 3, 2


def _moe_block_kernel(x_ref, obj_ref, gW_ref, gb_ref, sW1_ref, sb1_ref,
                      sW2_ref, sb2_ref, iW1_ref, ib1_ref, iW2_ref, ib2_ref,
                      out_ref):
    x = x_ref[...]
    T = x.shape[0]

    logits = jnp.dot(x, gW_ref[...], preferred_element_type=jnp.float32)
    logits = logits + gb_ref[0]
    iota = jax.lax.broadcasted_iota(jnp.int32, logits.shape, 1)
    m1 = jnp.max(logits, axis=1, keepdims=True)
    i1 = jnp.min(jnp.where(logits == m1, iota, _E_IND), axis=1, keepdims=True)
    l2 = jnp.where(iota == i1, -jnp.inf, logits)
    m2 = jnp.max(l2, axis=1, keepdims=True)
    i2 = jnp.min(jnp.where(l2 == m2, iota, _E_IND), axis=1, keepdims=True)
    w1 = 1.0 / (1.0 + jnp.exp(m2 - m1))
    w2 = 1.0 - w1

    obj = obj_ref[0]

    acc = jnp.zeros((T, out_ref.shape[1]), dtype=jnp.float32)
    for s in range(_E_SH):
        h = jnp.maximum(
            jnp.dot(x, sW1_ref[s], preferred_element_type=jnp.float32)
            + sb1_ref[s], 0.0)
        o = jnp.dot(h, sW2_ref[s], preferred_element_type=jnp.float32) \
            + sb2_ref[s]
        acc = acc + o * obj[:, s:s + 1]

    for e in range(_E_IND):
        ce = w1 * (i1 == e) + w2 * (i2 == e)
        h = jnp.maximum(
            jnp.dot(x, iW1_ref[e], preferred_element_type=jnp.float32)
            + ib1_ref[e], 0.0)
        o = jnp.dot(h, iW2_ref[e], preferred_element_type=jnp.float32) \
            + ib2_ref[e]
        acc = acc + o * ce

    out_ref[...] = 0.5 * acc


def kernel(feature_vectors, object_types, gW, gb, sW1, sb1, sW2, sb2,
           iW1, ib1, iW2, ib2):
    B, NN, HL, D = feature_vectors.shape
    N = B * NN * HL
    O = sW2.shape[-1]
    x = feature_vectors.reshape(N, D)
    T = 512 if N % 512 == 0 else (256 if N % 256 == 0 else N)
    nblk = N // T
    obj = object_types.reshape(nblk, T, 3)
    gb2 = gb.reshape(1, -1)

    full = lambda a: pl.BlockSpec(a.shape, lambda i: (0,) * a.ndim)
    out = pl.pallas_call(
        _moe_block_kernel,
        grid=(nblk,),
        in_specs=[
            pl.BlockSpec((T, D), lambda i: (i, 0)),
            pl.BlockSpec((1, T, 3), lambda i: (i, 0, 0)),
            full(gW), full(gb2), full(sW1), full(sb1), full(sW2), full(sb2),
            full(iW1), full(ib1), full(iW2), full(ib2),
        ],
        out_specs=pl.BlockSpec((T, O), lambda i: (i, 0)),
        out_shape=jax.ShapeDtypeStruct((N, O), jnp.float32),
        compiler_params=pltpu.CompilerParams(
            dimension_semantics=("parallel",)),
    )(x, obj, gW, gb2, sW1, sb1, sW2, sb2, iW1, ib1, iW2, ib2)
    return out.reshape(B, NN, HL, O)
